# Initial kernel scaffold; baseline (speedup 1.0000x reference)
import jax
import jax.numpy as jnp
from jax.experimental import pallas as pl


def kernel(x, edge_index, lin_W, lin_b, eps, W1, g1, b1, W2, g2, b2, W3, b3):
    raise NotImplementedError("write your pallas kernel here")


# speedup vs baseline: 5.4012x; 5.4012x over previous
import functools

import jax
import jax.numpy as jnp
from jax import lax
from jax.experimental import pallas as pl
from jax.experimental.pallas import tpu as pltpu
from jax.experimental.pallas import tpu_sc as plsc

_NC = 2
_NS = 16


def _sc_agg_body(npad, dh, nchunk, chunk, xh_hbm, src_hbm, dst_hbm, out_hbm,
                 idx_s, idx_d, rows, zbuf, acc, sem):
    c = lax.axis_index("c")
    s = lax.axis_index("s")
    rpt = npad // _NS
    zrows = zbuf.shape[0]

    def zero_row(i, carry):
        for l in range(dh // 16):
            zbuf[i, pl.ds(l * 16, 16)] = jnp.zeros((16,), jnp.float32)
        return carry
    lax.fori_loop(0, zrows, zero_row, 0)
    for j in range(rpt // zrows):
        pltpu.sync_copy(zbuf, acc.at[pl.ds(s * rpt + j * zrows, zrows)])
    plsc.subcore_barrier()

    pltpu.sync_copy(src_hbm.at[s], idx_s)
    pltpu.sync_copy(dst_hbm.at[s], idx_d)

    def chunk_body(i, carry):
        pltpu.async_copy(xh_hbm.at[c].at[idx_s.at[i]], rows, sem).wait()
        pltpu.sync_copy(rows, acc.at[idx_d.at[i]], add=True)
        return carry
    lax.fori_loop(0, nchunk, chunk_body, 0)
    plsc.subcore_barrier()

    pltpu.sync_copy(acc.at[pl.ds(s * rpt, rpt)],
                    out_hbm.at[c, pl.ds(s * rpt, rpt)])


def _sc_aggregate(xh, src, dst):
    _, n, dh = xh.shape
    e = src.shape[0]
    npad = -(-n // (128 * _NS)) * (128 * _NS)
    ept = e // _NS
    chunk = 125
    nchunk = ept // chunk
    src_r = src.reshape(_NS, nchunk, chunk)
    dst_r = dst.reshape(_NS, nchunk, chunk)
    mesh = plsc.VectorSubcoreMesh(core_axis_name="c", subcore_axis_name="s")
    return pl.kernel(
        functools.partial(_sc_agg_body, npad, dh, nchunk, chunk),
        out_type=jax.ShapeDtypeStruct((_NC, npad, dh), jnp.float32),
        mesh=mesh,
        scratch_types=[
            pltpu.VMEM((nchunk, chunk), jnp.int32),
            pltpu.VMEM((nchunk, chunk), jnp.int32),
            pltpu.VMEM((chunk, dh), jnp.float32),
            pltpu.VMEM((128, dh), jnp.float32),
            pltpu.VMEM_SHARED((npad, dh), jnp.float32),
            pltpu.SemaphoreType.DMA,
        ],
        compiler_params=pltpu.CompilerParams(use_tc_tiling_on_sc=False),
    )(xh, src_r, dst_r)


def _tc1_body(x_ref, a0_ref, a1_ref, wl_ref, bl_ref, w1_ref, ep_ref,
              h1_ref, sum_ref, sq_ref):
    xb = x_ref[...]
    agg = jnp.concatenate([a0_ref[...], a1_ref[...]], axis=-1)
    t = ep_ref[0, 0] * xb + agg
    h = jnp.dot(xb, wl_ref[...], preferred_element_type=jnp.float32)
    h = h + bl_ref[...] + jnp.concatenate([t, t], axis=-1)
    h1 = jnp.dot(h, w1_ref[...], preferred_element_type=jnp.float32)
    h1_ref[...] = h1
    s = jnp.sum(h1, axis=0, keepdims=True)
    q = jnp.sum(h1 * h1, axis=0, keepdims=True)

    @pl.when(pl.program_id(0) == 0)
    def _():
        sum_ref[...] = s
        sq_ref[...] = q

    @pl.when(pl.program_id(0) != 0)
    def _():
        sum_ref[...] += s
        sq_ref[...] += q


def _tc2_body(h_ref, sc_ref, sh_ref, w_ref, o_ref, sum_ref, sq_ref):
    a = jnp.maximum(h_ref[...] * sc_ref[...] + sh_ref[...], 0.0)
    o = jnp.dot(a, w_ref[...], preferred_element_type=jnp.float32)
    o_ref[...] = o
    s = jnp.sum(o, axis=0, keepdims=True)
    q = jnp.sum(o * o, axis=0, keepdims=True)

    @pl.when(pl.program_id(0) == 0)
    def _():
        sum_ref[...] = s
        sq_ref[...] = q

    @pl.when(pl.program_id(0) != 0)
    def _():
        sum_ref[...] += s
        sq_ref[...] += q


def _tc3_body(h_ref, sc_ref, sh_ref, w_ref, b_ref, o_ref):
    a = jnp.maximum(h_ref[...] * sc_ref[...] + sh_ref[...], 0.0)
    o_ref[...] = (jnp.dot(a, w_ref[...], preferred_element_type=jnp.float32)
                  + b_ref[...])


def _bn_coeffs(ssum, ssq, n, g, b):
    mean = ssum / n
    var = ssq / n - mean * mean
    scale = g[None, :] / jnp.sqrt(var + 1e-5)
    shift = b[None, :] - mean * scale
    return scale, shift


def kernel(x, edge_index, lin_W, lin_b, eps, W1, g1, b1, W2, g2, b2, W3, b3):
    n, d = x.shape
    h = lin_W.shape[1]
    dh = d // 2
    xh = jnp.stack([x[:, :dh], x[:, dh:]])
    agg2 = _sc_aggregate(xh, edge_index[0], edge_index[1])

    blk = 1000
    grid = (n // blk,)
    epp1 = (1.0 + eps).reshape(1, 1).astype(jnp.float32)

    full = lambda shape: pl.BlockSpec(shape, lambda i: (0,) * len(shape))
    rows_in = pl.BlockSpec((blk, d), lambda i: (i, 0))
    rows_half = pl.BlockSpec((blk, dh), lambda i: (i, 0))
    rows_h = pl.BlockSpec((blk, h), lambda i: (i, 0))

    h1, s1, q1 = pl.pallas_call(
        _tc1_body,
        grid=grid,
        in_specs=[rows_in, rows_half, rows_half, full((d, h)), full((1, h)),
                  full((h, h)), full((1, 1))],
        out_specs=[rows_h, full((1, h)), full((1, h))],
        out_shape=[jax.ShapeDtypeStruct((n, h), jnp.float32),
                   jax.ShapeDtypeStruct((1, h), jnp.float32),
                   jax.ShapeDtypeStruct((1, h), jnp.float32)],
    )(x, agg2[0], agg2[1], lin_W, lin_b.reshape(1, h), W1, epp1)

    sc1, sh1 = _bn_coeffs(s1, q1, n, g1, b1)

    h2, s2, q2 = pl.pallas_call(
        _tc2_body,
        grid=grid,
        in_specs=[rows_h, full((1, h)), full((1, h)), full((h, h))],
        out_specs=[rows_h, full((1, h)), full((1, h))],
        out_shape=[jax.ShapeDtypeStruct((n, h), jnp.float32),
                   jax.ShapeDtypeStruct((1, h), jnp.float32),
                   jax.ShapeDtypeStruct((1, h), jnp.float32)],
    )(h1, sc1, sh1, W2)

    sc2, sh2 = _bn_coeffs(s2, q2, n, g2, b2)

    out = pl.pallas_call(
        _tc3_body,
        grid=grid,
        in_specs=[rows_h, full((1, h)), full((1, h)), full((h, d)),
                  full((1, d))],
        out_specs=rows_in,
        out_shape=jax.ShapeDtypeStruct((n, d), jnp.float32),
    )(h2, sc2, sh2, W3, b3.reshape(1, d))
    return out


# speedup vs baseline: 8.3061x; 1.5378x over previous
import functools

import jax
import jax.numpy as jnp
from jax import lax
from jax.experimental import pallas as pl
from jax.experimental.pallas import tpu as pltpu
from jax.experimental.pallas import tpu_sc as plsc

_NC = 2
_NS = 16


def _sc_agg_body(npad, dh, nchunk, chunk, xh_hbm, src_hbm, dst_hbm, out_hbm,
                 idx_s, idx_d, rows, zbuf, acc, sem0, sem1):
    c = lax.axis_index("c")
    s = lax.axis_index("s")
    rpt = npad // _NS
    zrows = zbuf.shape[0]
    sems = (sem0, sem1)

    def zero_row(i, carry):
        for l in range(dh // 16):
            zbuf[i, pl.ds(l * 16, 16)] = jnp.zeros((16,), jnp.float32)
        return carry
    lax.fori_loop(0, zrows, zero_row, 0)
    for j in range(rpt // zrows):
        pltpu.sync_copy(zbuf, acc.at[pl.ds(s * rpt + j * zrows, zrows)])

    pltpu.sync_copy(src_hbm.at[s], idx_s)
    pltpu.sync_copy(dst_hbm.at[s], idx_d)

    def start_gather(k, b):
        pltpu.async_copy(xh_hbm.at[c].at[idx_s.at[k]], rows.at[b], sems[b])

    def wait_gather(b):
        pltpu.make_async_copy(xh_hbm.at[c, pl.ds(0, chunk)],
                              rows.at[b], sems[b]).wait()

    def scatter(k, b):
        pltpu.sync_copy(rows.at[b], acc.at[idx_d.at[k]], add=True)

    start_gather(0, 0)
    start_gather(1, 1)
    plsc.subcore_barrier()

    def outer(k, carry):
        a = 2 * k
        wait_gather(0)
        scatter(a, 0)
        start_gather(a + 2, 0)
        wait_gather(1)
        scatter(a + 1, 1)
        start_gather(a + 3, 1)
        return carry
    lax.fori_loop(0, nchunk // 2 - 1, outer, 0)

    wait_gather(0)
    scatter(nchunk - 2, 0)
    wait_gather(1)
    scatter(nchunk - 1, 1)
    plsc.subcore_barrier()

    pltpu.sync_copy(acc.at[pl.ds(s * rpt, rpt)],
                    out_hbm.at[c, pl.ds(s * rpt, rpt)])


def _sc_aggregate(xh, src, dst):
    _, n, dh = xh.shape
    e = src.shape[0]
    npad = -(-n // (128 * _NS)) * (128 * _NS)
    ept = e // _NS
    chunk = 125
    nchunk = ept // chunk
    src_r = src.reshape(_NS, nchunk, chunk)
    dst_r = dst.reshape(_NS, nchunk, chunk)
    mesh = plsc.VectorSubcoreMesh(core_axis_name="c", subcore_axis_name="s")
    return pl.kernel(
        functools.partial(_sc_agg_body, npad, dh, nchunk, chunk),
        out_type=jax.ShapeDtypeStruct((_NC, npad, dh), jnp.float32),
        mesh=mesh,
        scratch_types=[
            pltpu.VMEM((nchunk, chunk), jnp.int32),
            pltpu.VMEM((nchunk, chunk), jnp.int32),
            pltpu.VMEM((2, chunk, dh), jnp.float32),
            pltpu.VMEM((128, dh), jnp.float32),
            pltpu.VMEM_SHARED((npad, dh), jnp.float32),
            pltpu.SemaphoreType.DMA,
            pltpu.SemaphoreType.DMA,
        ],
        compiler_params=pltpu.CompilerParams(use_tc_tiling_on_sc=False),
    )(xh, src_r, dst_r)


def _tc_body(nblk, blk, x_ref, a0_ref, a1_ref, wl_ref, bl_ref, w1_ref,
             w2_ref, w3_ref, b3_ref, ep_ref, g1_ref, be1_ref, g2_ref,
             be2_ref, o_ref, h1_s, h2_s, s1, q1, s2, q2, co1, co2):
    p = pl.program_id(0)
    i = pl.program_id(1)
    n = nblk * blk

    def stats_accum(v, s_ref, q_ref):
        s = jnp.sum(v, axis=0, keepdims=True)
        q = jnp.sum(v * v, axis=0, keepdims=True)

        @pl.when(i == 0)
        def _():
            s_ref[...] = s
            q_ref[...] = q

        @pl.when(i != 0)
        def _():
            s_ref[...] += s
            q_ref[...] += q

    def bn_coeffs(s_ref, q_ref, g_ref, b_ref, co_ref):
        mean = s_ref[...] / n
        var = q_ref[...] / n - mean * mean
        scale = g_ref[...] / jnp.sqrt(var + 1e-5)
        co_ref[0:1, :] = scale
        co_ref[1:2, :] = b_ref[...] - mean * scale

    @pl.when(p == 0)
    def _():
        xb = x_ref[...]
        agg = jnp.concatenate([a0_ref[...], a1_ref[...]], axis=-1)
        t = ep_ref[0, 0] * xb + agg
        hh = jnp.dot(xb, wl_ref[...], preferred_element_type=jnp.float32)
        hh = hh + bl_ref[...] + jnp.concatenate([t, t], axis=-1)
        h1 = jnp.dot(hh, w1_ref[...], preferred_element_type=jnp.float32)
        h1_s[pl.ds(i * blk, blk), :] = h1
        stats_accum(h1, s1, q1)

    @pl.when(p == 1)
    def _():
        @pl.when(i == 0)
        def _():
            bn_coeffs(s1, q1, g1_ref, be1_ref, co1)
        h1 = h1_s[pl.ds(i * blk, blk), :]
        a = jnp.maximum(h1 * co1[0:1, :] + co1[1:2, :], 0.0)
        h2 = jnp.dot(a, w2_ref[...], preferred_element_type=jnp.float32)
        h2_s[pl.ds(i * blk, blk), :] = h2
        stats_accum(h2, s2, q2)

    @pl.when(p == 2)
    def _():
        @pl.when(i == 0)
        def _():
            bn_coeffs(s2, q2, g2_ref, be2_ref, co2)
        h2 = h2_s[pl.ds(i * blk, blk), :]
        a = jnp.maximum(h2 * co2[0:1, :] + co2[1:2, :], 0.0)
        o_ref[...] = (jnp.dot(a, w3_ref[...],
                              preferred_element_type=jnp.float32)
                      + b3_ref[...])


def kernel(x, edge_index, lin_W, lin_b, eps, W1, g1, b1, W2, g2, b2, W3, b3):
    n, d = x.shape
    h = lin_W.shape[1]
    dh = d // 2
    xh = jnp.stack([x[:, :dh], x[:, dh:]])
    agg2 = _sc_aggregate(xh, edge_index[0], edge_index[1])

    blk = 1000
    nblk = n // blk
    grid = (3, nblk)
    epp1 = (1.0 + eps).reshape(1, 1).astype(jnp.float32)

    def full(shape):
        return pl.BlockSpec(shape, lambda p, i: (0,) * len(shape))

    def rows_spec(cols):
        return pl.BlockSpec((blk, cols),
                            lambda p, i: (jnp.where(p == 0, i, 0), 0))

    out_spec = pl.BlockSpec((blk, d), lambda p, i: (jnp.where(p == 2, i, 0), 0))

    vec_h = full((1, h))
    out = pl.pallas_call(
        functools.partial(_tc_body, nblk, blk),
        grid=grid,
        in_specs=[rows_spec(d), rows_spec(dh), rows_spec(dh), full((d, h)),
                  vec_h, full((h, h)), full((h, h)), full((h, d)),
                  full((1, d)), full((1, 1)), vec_h, vec_h, vec_h, vec_h],
        out_specs=out_spec,
        out_shape=jax.ShapeDtypeStruct((n, d), jnp.float32),
        scratch_shapes=[
            pltpu.VMEM((n, h), jnp.float32),
            pltpu.VMEM((n, h), jnp.float32),
            pltpu.VMEM((1, h), jnp.float32),
            pltpu.VMEM((1, h), jnp.float32),
            pltpu.VMEM((1, h), jnp.float32),
            pltpu.VMEM((1, h), jnp.float32),
            pltpu.VMEM((2, h), jnp.float32),
            pltpu.VMEM((2, h), jnp.float32),
        ],
    )(x, agg2[0], agg2[1], lin_W, lin_b.reshape(1, h), W1, W2, W3,
      b3.reshape(1, d), epp1, g1.reshape(1, h), b1.reshape(1, h),
      g2.reshape(1, h), b2.reshape(1, h))
    return out
